# Initial kernel scaffold; baseline (speedup 1.0000x reference)
import jax
import jax.numpy as jnp
from jax.experimental import pallas as pl


def kernel(x, edge_index, W1, b1, W2, b2, Wfc, bfc):
    raise NotImplementedError("write your pallas kernel here")


# speedup vs baseline: 50.4692x; 50.4692x over previous
import functools

import jax
import jax.numpy as jnp
from jax import lax
from jax.experimental import pallas as pl
from jax.experimental.pallas import tpu as pltpu
from jax.experimental.pallas import tpu_sc as plsc

N = 100000
NP = 102400
E = 1600000
WIN = 128
NWIN = 12512
E1 = WIN * NWIN
NC, NS = 2, 16
RP = NP // NS
H1, H2, OUT = 64, 32, 32
FH = 16
F1 = 8

_mesh = plsc.VectorSubcoreMesh(core_axis_name="c", subcore_axis_name="s")
_sc_params = pltpu.CompilerParams(use_tc_tiling_on_sc=False)
f32 = jnp.float32


def _fill(ref, n, value):
    @pl.loop(0, n, step=16)
    def _(i):
        ref[pl.ds(i, 16)] = jnp.full((16,), value, f32)


def _hist_body(dst_hbm, zeros_hbm, out_hbm, acc, ones):
    c = lax.axis_index("c")
    s = lax.axis_index("s")
    _fill(ones, WIN, 1.0)
    srow = pl.multiple_of(s * RP, 128)
    pltpu.sync_copy(zeros_hbm, acc.at[pl.ds(srow, RP)])
    plsc.subcore_barrier()

    def body(didx):
        pltpu.sync_copy(ones, acc.at[didx.at[0]], add=True)

    pltpu.emit_pipeline(
        body,
        grid=(NWIN,),
        in_specs=[pl.BlockSpec((1, WIN), lambda i: (0, i))],
        out_specs=[],
        core_axis_name=("c", "s"),
        dimension_semantics=(pltpu.PARALLEL,),
    )(dst_hbm)
    plsc.subcore_barrier()
    gof = pl.multiple_of(c * NP + srow, 128)
    pltpu.sync_copy(acc.at[pl.ds(srow, RP)], out_hbm.at[pl.ds(gof, RP)])


_sc_hist = pl.kernel(
    _hist_body,
    out_type=jax.ShapeDtypeStruct((NC * NP,), f32),
    mesh=_mesh,
    scratch_types=[
        pltpu.VMEM_SHARED((NP,), f32),
        pltpu.VMEM((WIN,), f32),
    ],
    compiler_params=_sc_params,
)


def _agg1_body(src_hbm, dst_hbm, tab_hbm, zeros_hbm, out_hbm, acc, rows):
    c = lax.axis_index("c")
    s = lax.axis_index("s")
    srow = pl.multiple_of(s * RP, 128)
    pltpu.sync_copy(zeros_hbm, acc.at[pl.ds(srow, RP)])
    plsc.subcore_barrier()

    def body(sidx, didx):
        pltpu.sync_copy(tab_hbm.at[sidx.at[0]], rows)
        pltpu.sync_copy(rows, acc.at[didx.at[0]], add=True)

    pltpu.emit_pipeline(
        body,
        grid=(NWIN,),
        in_specs=[
            pl.BlockSpec((1, WIN), lambda i: (0, i)),
            pl.BlockSpec((1, WIN), lambda i: (0, i)),
        ],
        out_specs=[],
        core_axis_name=("c", "s"),
        dimension_semantics=(pltpu.PARALLEL,),
    )(src_hbm, dst_hbm)
    plsc.subcore_barrier()
    pltpu.sync_copy(acc.at[pl.ds(srow, RP)], out_hbm.at[c, pl.ds(srow, RP)])


_sc_agg1 = pl.kernel(
    _agg1_body,
    out_type=jax.ShapeDtypeStruct((NC, NP, F1), f32),
    mesh=_mesh,
    scratch_types=[
        pltpu.VMEM_SHARED((NP, F1), f32),
        pltpu.VMEM((WIN, F1), f32),
    ],
    compiler_params=_sc_params,
)


def _agg2_body(src_hbm, dst_hbm, tab_hbm, zeros_hbm, out_hbm, acc, rows, sadj):
    c = lax.axis_index("c")
    s = lax.axis_index("s")
    srow = pl.multiple_of(s * RP, 128)
    pltpu.sync_copy(zeros_hbm, acc.at[pl.ds(srow, RP)])
    plsc.subcore_barrier()
    off = c * NP

    def body(sidx, didx):
        for k in range(WIN // 16):
            sadj[pl.ds(k * 16, 16)] = sidx[0, pl.ds(k * 16, 16)] + off
        pltpu.sync_copy(tab_hbm.at[sadj], rows)
        pltpu.sync_copy(rows, acc.at[didx.at[0]], add=True)

    pltpu.emit_pipeline(
        body,
        grid=(NWIN,),
        in_specs=[
            pl.BlockSpec((1, WIN), lambda i: (0, i)),
            pl.BlockSpec((1, WIN), lambda i: (0, i)),
        ],
        out_specs=[],
        core_axis_name="s",
        dimension_semantics=(pltpu.PARALLEL,),
    )(src_hbm, dst_hbm)
    plsc.subcore_barrier()
    pltpu.sync_copy(acc.at[pl.ds(srow, RP)], out_hbm.at[c, pl.ds(srow, RP)])


_sc_agg2 = pl.kernel(
    _agg2_body,
    out_type=jax.ShapeDtypeStruct((NC, NP, FH), f32),
    mesh=_mesh,
    scratch_types=[
        pltpu.VMEM_SHARED((NP, FH), f32),
        pltpu.VMEM((WIN, FH), f32),
        pltpu.VMEM((WIN,), jnp.int32),
    ],
    compiler_params=_sc_params,
)

R_TC = 3200
GRID_TC = NP // R_TC


def _tc1_body(h0, h1, x, u1, dinv):
    deg = 1.0 + h0[...] + h1[...]
    di = lax.rsqrt(deg)
    dinv[...] = di
    u1[...] = jnp.pad(x[...] * di, ((0, 0), (0, F1 - 4)))


def _tc1(h0, h1, x_p):
    return pl.pallas_call(
        _tc1_body,
        grid=(GRID_TC,),
        in_specs=[
            pl.BlockSpec((R_TC, 1), lambda i: (i, 0)),
            pl.BlockSpec((R_TC, 1), lambda i: (i, 0)),
            pl.BlockSpec((R_TC, 4), lambda i: (i, 0)),
        ],
        out_specs=[
            pl.BlockSpec((R_TC, F1), lambda i: (i, 0)),
            pl.BlockSpec((R_TC, 1), lambda i: (i, 0)),
        ],
        out_shape=[
            jax.ShapeDtypeStruct((NP, F1), f32),
            jax.ShapeDtypeStruct((NP, 1), f32),
        ],
    )(h0, h1, x_p)


def _tc2_body(a0, a1, u1, dinv, w1, b1, w2, g2):
    t = (a0[...] + a1[...] + u1[...])[:, :4]
    di = dinv[...]
    h = jnp.maximum(jnp.dot(t, w1[...], preferred_element_type=f32) * di
                    + b1[...][None, :], 0.0)
    g = jnp.dot(h, w2[...], preferred_element_type=f32) * di
    g2[0] = g[:, :FH]
    g2[1] = g[:, FH:]


def _tc2(a0, a1, u1, dinv, W1, b1, W2):
    return pl.pallas_call(
        _tc2_body,
        grid=(GRID_TC,),
        in_specs=[
            pl.BlockSpec((R_TC, F1), lambda i: (i, 0)),
            pl.BlockSpec((R_TC, F1), lambda i: (i, 0)),
            pl.BlockSpec((R_TC, F1), lambda i: (i, 0)),
            pl.BlockSpec((R_TC, 1), lambda i: (i, 0)),
            pl.BlockSpec((4, H1), lambda i: (0, 0)),
            pl.BlockSpec((H1,), lambda i: (0,)),
            pl.BlockSpec((H1, H2), lambda i: (0, 0)),
        ],
        out_specs=pl.BlockSpec((NC, R_TC, FH), lambda i: (0, i, 0)),
        out_shape=jax.ShapeDtypeStruct((NC, NP, FH), f32),
    )(a0, a1, u1, dinv, W1, b1, W2)


def _tc3_body(aa, ab, ga, gb, dinv, b2, wfc, bfc, y):
    t = jnp.concatenate([aa[...] + ga[...], ab[...] + gb[...]], axis=1)
    di = dinv[...]
    o2 = jnp.maximum(t * di + b2[...][None, :], 0.0)
    y[...] = jnp.maximum(
        jnp.dot(o2, wfc[...], preferred_element_type=f32) + bfc[...][None, :], 0.0
    )


def _tc3(aa, ab, ga, gb, dinv, b2, Wfc, bfc):
    return pl.pallas_call(
        _tc3_body,
        grid=(GRID_TC,),
        in_specs=[
            pl.BlockSpec((R_TC, FH), lambda i: (i, 0)),
            pl.BlockSpec((R_TC, FH), lambda i: (i, 0)),
            pl.BlockSpec((R_TC, FH), lambda i: (i, 0)),
            pl.BlockSpec((R_TC, FH), lambda i: (i, 0)),
            pl.BlockSpec((R_TC, 1), lambda i: (i, 0)),
            pl.BlockSpec((H2,), lambda i: (0,)),
            pl.BlockSpec((H2, OUT), lambda i: (0, 0)),
            pl.BlockSpec((OUT,), lambda i: (0,)),
        ],
        out_specs=pl.BlockSpec((R_TC, OUT), lambda i: (i, 0)),
        out_shape=jax.ShapeDtypeStruct((NP, OUT), f32),
    )(aa, ab, ga, gb, dinv, b2, Wfc, bfc)


def kernel(x, edge_index, W1, b1, W2, b2, Wfc, bfc):
    ei = edge_index.astype(jnp.int32)
    src, dst = ei[0], ei[1]
    npad = E1 - E
    pad_idx = N + (jnp.arange(npad, dtype=jnp.int32) % (NP - N))
    src_p = jnp.concatenate([src, pad_idx]).reshape(1, E1)
    dst_p = jnp.concatenate([dst, pad_idx]).reshape(1, E1)
    x_p = jnp.pad(x, ((0, NP - N), (0, 0)))

    z1 = jnp.zeros((RP,), f32)
    z4 = jnp.zeros((RP, F1), f32)
    z16 = jnp.zeros((RP, FH), f32)

    hist = _sc_hist(dst_p, z1)
    u1, dinv = _tc1(hist[:NP].reshape(NP, 1), hist[NP:].reshape(NP, 1), x_p)
    agg1 = _sc_agg1(src_p, dst_p, u1, z4)
    G2 = _tc2(agg1[0], agg1[1], u1, dinv, W1, b1, W2)
    A2 = _sc_agg2(src_p, dst_p, G2.reshape(NC * NP, FH), z16)
    y = _tc3(A2[0], A2[1], G2[0], G2[1], dinv, b2, Wfc, bfc)
    return y[:N]
